# Initial kernel scaffold; baseline (speedup 1.0000x reference)
import jax
import jax.numpy as jnp
from jax.experimental import pallas as pl


def kernel(x, edge_index, W):
    raise NotImplementedError("write your pallas kernel here")


# speedup vs baseline: 1.0365x; 1.0365x over previous
import functools

import jax
import jax.numpy as jnp
from jax import lax
from jax.experimental import pallas as pl
from jax.experimental.pallas import tpu as pltpu
from jax.experimental.pallas import tpu_sc as plsc

N_NODES = 10000
D_FEAT = 256
D_LATENT = 128
N_EDGES = 160000

_NC = 2
_NS = 16
_NW = _NC * _NS
_L = 16

_CHUNK = 128
_NCHUNKS = N_EDGES // _CHUNK
_CPW = -(-_NCHUNKS // _NW)


def _encode_matmul(x, W):
    M, K = x.shape
    _, N = W.shape
    BM = 2000

    def body(x_ref, w_ref, z_ref):
        z_ref[...] = jnp.dot(x_ref[...], w_ref[...],
                             preferred_element_type=jnp.float32)

    return pl.pallas_call(
        body,
        grid=(M // BM,),
        in_specs=[
            pl.BlockSpec((BM, K), lambda i: (i, 0)),
            pl.BlockSpec((K, N), lambda i: (0, 0)),
        ],
        out_specs=pl.BlockSpec((BM, N), lambda i: (i, 0)),
        out_shape=jax.ShapeDtypeStruct((M, N), jnp.float32),
    )(x, W)


def _decode_body(z_hbm, src_hbm, dst_hbm, out_hbm,
                 idx_s, idx_d, rows_s, rows_d, out_v, sem_s, sem_d):
    wid = lax.axis_index("s") * _NC + lax.axis_index("c")

    def chunk_body(i, carry):
        c = wid + i * _NW

        @pl.when(c < _NCHUNKS)
        def _():
            base = c * _CHUNK
            pltpu.sync_copy(src_hbm.at[pl.ds(base, _CHUNK)], idx_s)
            pltpu.sync_copy(dst_hbm.at[pl.ds(base, _CHUNK)], idx_d)
            cp_s = pltpu.async_copy(z_hbm.at[idx_s], rows_s, sem_s)
            cp_d = pltpu.async_copy(z_hbm.at[idx_d], rows_d, sem_d)
            cp_s.wait()
            cp_d.wait()
            for g in range(_CHUNK // _L):
                rowv = lax.iota(jnp.int32, _L) + g * _L

                def col_body(j, acc):
                    jv = jnp.full((_L,), j, jnp.int32)
                    sv = plsc.load_gather(rows_s, [rowv, jv])
                    dv = plsc.load_gather(rows_d, [rowv, jv])
                    return acc + sv * dv

                acc = lax.fori_loop(0, D_LATENT, col_body,
                                    jnp.zeros((_L,), jnp.float32))
                e = jnp.exp(-jnp.abs(acc))
                sig = jnp.where(acc >= 0.0, 1.0 / (1.0 + e), e / (1.0 + e))
                out_v[pl.ds(g * _L, _L)] = sig
            pltpu.sync_copy(out_v, out_hbm.at[pl.ds(base, _CHUNK)])

        return carry

    lax.fori_loop(0, _CPW, chunk_body, 0)


def _decode(z, src, dst):
    mesh = plsc.VectorSubcoreMesh(core_axis_name="c", subcore_axis_name="s")
    k = functools.partial(
        pl.kernel,
        out_type=jax.ShapeDtypeStruct((N_EDGES,), jnp.float32),
        mesh=mesh,
        scratch_types=[
            pltpu.VMEM((_CHUNK,), jnp.int32),
            pltpu.VMEM((_CHUNK,), jnp.int32),
            pltpu.VMEM((_CHUNK, D_LATENT), jnp.float32),
            pltpu.VMEM((_CHUNK, D_LATENT), jnp.float32),
            pltpu.VMEM((_CHUNK,), jnp.float32),
            pltpu.SemaphoreType.DMA,
            pltpu.SemaphoreType.DMA,
        ],
        compiler_params=pltpu.CompilerParams(needs_layout_passes=False),
    )(_decode_body)
    return k(z, src, dst)


def kernel(x, edge_index, W):
    z = _encode_matmul(x, W)
    ei = edge_index.astype(jnp.int32)
    return _decode(z, ei[0], ei[1])
